# Initial kernel scaffold; baseline (speedup 1.0000x reference)
import jax
import jax.numpy as jnp
from jax.experimental import pallas as pl


def kernel(src_neurons, cached_targets, weights):
    raise NotImplementedError("write your pallas kernel here")


# speedup vs baseline: 1.4131x; 1.4131x over previous
import functools

import jax
import jax.numpy as jnp
from jax import lax
from jax.experimental import pallas as pl
from jax.experimental.pallas import tpu as pltpu
from jax.experimental.pallas import tpu_sc as plsc

_B = 16384
_D = 32
_NSRC = 10000

_info = plsc.get_sparse_core_info()
_NC = _info.num_cores
_NS = _info.num_subcores
_NW = _NC * _NS
_BPW = _B // _NW
_CH = 128
_NCH = _BPW // _CH


@functools.partial(
    pl.kernel,
    out_type=(
        jax.ShapeDtypeStruct((_B, _D), jnp.int32),
        jax.ShapeDtypeStruct((_B, _D), jnp.float32),
    ),
    mesh=plsc.VectorSubcoreMesh(core_axis_name="c", subcore_axis_name="s"),
    scratch_types=[
        pltpu.VMEM((_NCH, _CH), jnp.int32),
        pltpu.VMEM((_BPW, _D), jnp.int32),
        pltpu.VMEM((_BPW, _D), jnp.float32),
        pltpu.SemaphoreType.DMA,
        pltpu.SemaphoreType.DMA,
    ],
    compiler_params=pltpu.CompilerParams(use_tc_tiling_on_sc=False),
)
def _gather2(idx_hbm, tgt_hbm, w_hbm, out_t, out_w,
             idx_v, t_rows, w_rows, sem_t, sem_w):
    wid = lax.axis_index("s") * _NC + lax.axis_index("c")
    base = wid * _BPW
    pltpu.sync_copy(idx_hbm.at[pl.ds(wid * _NCH, _NCH)], idx_v)
    copies = []
    for c in range(_NCH):
        copies.append(pltpu.async_copy(
            tgt_hbm.at[idx_v.at[c]], t_rows.at[pl.ds(c * _CH, _CH)], sem_t))
        copies.append(pltpu.async_copy(
            w_hbm.at[idx_v.at[c]], w_rows.at[pl.ds(c * _CH, _CH)], sem_w))
    for cp in copies:
        cp.wait()
    pltpu.sync_copy(t_rows, out_t.at[pl.ds(base, _BPW)])
    pltpu.sync_copy(w_rows, out_w.at[pl.ds(base, _BPW)])


def kernel(src_neurons, cached_targets, weights):
    idx2d = src_neurons.astype(jnp.int32).reshape(_NW * _NCH, _CH)
    return _gather2(idx2d, cached_targets, weights)
